# baseline (speedup 1.0000x reference)
import jax
import jax.numpy as jnp
from jax.experimental import pallas as pl
from jax.experimental.pallas import tpu as pltpu

_N = 5000
_SIGMA = 0.5
_NP = 5120
_BMT = 1024
_NI = _NP // _BMT
_NT = _NI * (_NI + 1) // 2


def _tri_body(bi_ref, bj_ref, pi_r, pj_r, sc3_r, out_r,
              acc_r, row_r, col_r):
    t = pl.program_id(0)
    bi = bi_ref[t]
    bj = bj_ref[t]

    @pl.when(t == 0)
    def _():
        col_r[...] = jnp.zeros_like(col_r)

    @pl.when(bj == 0)
    def _():
        acc_r[...] = jnp.zeros_like(acc_r)

    pj = jnp.swapaxes(pj_r[...], 0, 1)
    sci = pi_r[:, 5:6]
    scj = pj[5:6, :]
    xx1 = jnp.maximum(pi_r[:, 0:1], pj[0:1, :])
    yy1 = jnp.maximum(pi_r[:, 1:2], pj[1:2, :])
    xx2 = jnp.minimum(pi_r[:, 2:3], pj[2:3, :])
    yy2 = jnp.minimum(pi_r[:, 3:4], pj[3:4, :])
    w = jnp.maximum(xx2 - xx1, 0.0)
    h = jnp.maximum(yy2 - yy1, 0.0)
    inter = w * h
    union = (pi_r[:, 4:5] + pj[4:5, :]) - inter
    iou = inter / union
    q = iou * iou
    ones_col = jnp.ones((_BMT, 1), jnp.float32)
    acc_r[...] += jnp.dot(jnp.where(scj > sci, q, 0.0), ones_col,
                          preferred_element_type=jnp.float32)

    @pl.when(bj < bi)
    def _():
        ones_row = jnp.ones((1, _BMT), jnp.float32)
        cs = jnp.dot(ones_row, jnp.where(sci > scj, q, 0.0),
                     preferred_element_type=jnp.float32)
        col_r[bj] += cs

    @pl.when(bj == bi)
    def _():
        row_r[bi] = jnp.swapaxes(acc_r[...], 0, 1)

    @pl.when(t == _NT - 1)
    def _():
        out_r[...] = sc3_r[...] * jnp.exp(
            (row_r[...] + col_r[...]) * (-1.0 / _SIGMA))


def _soft_nms(p, pt, sc3):
    steps = [(bi, bj) for bi in range(_NI) for bj in range(bi + 1)]
    bi_arr = jnp.array([s[0] for s in steps], jnp.int32)
    bj_arr = jnp.array([s[1] for s in steps], jnp.int32)
    grid_spec = pltpu.PrefetchScalarGridSpec(
        num_scalar_prefetch=2,
        grid=(len(steps),),
        in_specs=[
            pl.BlockSpec((_BMT, 8), lambda t, bi, bj: (bi[t], 0)),
            pl.BlockSpec((_BMT, 8), lambda t, bi, bj: (bj[t], 0)),
            pl.BlockSpec((_NI, 1, _BMT), lambda t, bi, bj: (0, 0, 0)),
        ],
        out_specs=pl.BlockSpec((_NI, 1, _BMT), lambda t, bi, bj: (0, 0, 0)),
        scratch_shapes=[
            pltpu.VMEM((_BMT, 1), jnp.float32),
            pltpu.VMEM((_NI, 1, _BMT), jnp.float32),
            pltpu.VMEM((_NI, 1, _BMT), jnp.float32),
        ],
    )
    new = pl.pallas_call(
        _tri_body,
        grid_spec=grid_spec,
        out_shape=jax.ShapeDtypeStruct((_NI, 1, _BMT), jnp.float32),
        compiler_params=pltpu.CompilerParams(
            dimension_semantics=("arbitrary",)
        ),
    )(bi_arr, bj_arr, p, pt, sc3)
    return new.reshape(_NP)


def kernel(boxes, scores):
    pad = _NP - _N
    x1, y1, x2, y2 = boxes[:, 0], boxes[:, 1], boxes[:, 2], boxes[:, 3]
    ar = (x2 - x1) * (y2 - y1)
    p = jnp.stack([x1, y1, x2, y2, ar, scores,
                   jnp.zeros_like(ar), jnp.zeros_like(ar)], axis=1)
    prow = jnp.array([[0.0, 0.0, 1.0, 1.0, 1.0, -jnp.inf, 0.0, 0.0]],
                     jnp.float32)
    p = jnp.concatenate([p, jnp.broadcast_to(prow, (pad, 8))], axis=0)
    sc3 = p[:, 5].reshape(_NI, 1, _BMT)
    out = _soft_nms(p, p, sc3)
    return out[:_N]
